# baseline (speedup 1.0000x reference)
import functools

import jax
import jax.numpy as jnp
from jax import lax
from jax.experimental import pallas as pl
from jax.experimental.pallas import tpu as pltpu
from jax.experimental.pallas import tpu_sc as plsc

EMB = 64
LANES = 16
NC = 2
NS = 16
NW = NC * NS


WID = 128


def _build_sc_call(B, L):
    assert B % (NW * 32) == 0 and EMB % LANES == 0
    bpw = B // NW
    CH = 16
    n_chunks = bpw // CH
    rows_per_chunk = CH * L
    KV = EMB // LANES

    mesh = plsc.VectorSubcoreMesh(core_axis_name="c", subcore_axis_name="s")

    @functools.partial(
        pl.kernel,
        mesh=mesh,
        compiler_params=pltpu.CompilerParams(use_tc_tiling_on_sc=False),
        out_type=jax.ShapeDtypeStruct((B,), jnp.float32),
        scratch_types=[
            pltpu.VMEM((bpw,), jnp.int32),
            pltpu.VMEM((bpw,), jnp.int32),
            pltpu.VMEM((bpw * L,), jnp.int32),
            pltpu.VMEM((bpw,), jnp.float32),
            pltpu.VMEM((bpw,), jnp.float32),
            pltpu.VMEM((2, CH, WID), jnp.float32),
            pltpu.VMEM((2, CH, WID), jnp.float32),
            pltpu.VMEM((2, rows_per_chunk, WID), jnp.float32),
            pltpu.VMEM((bpw,), jnp.float32),
            pltpu.SemaphoreType.DMA,
            pltpu.SemaphoreType.DMA,
            pltpu.SemaphoreType.DMA,
        ],
    )
    def body(u_hbm, i_hbm, a_hbm, n_hbm, ut_wide, at_wide, it_wide, bt,
             out_hbm, uidx, iidx, aidx, n_v, bias_v, ubuf, ibuf, abuf,
             out_v, sem0, sem1, sem_s):
        ut, at, it = ut_wide, at_wide, it_wide
        wid = lax.axis_index("s") * NC + lax.axis_index("c")
        base = wid * bpw
        st = [
            pltpu.async_copy(u_hbm.at[pl.ds(base, bpw)], uidx, sem_s),
            pltpu.async_copy(i_hbm.at[pl.ds(base, bpw)], iidx, sem_s),
            pltpu.async_copy(a_hbm.at[pl.ds(base * L, bpw * L)], aidx, sem_s),
            pltpu.async_copy(n_hbm.at[pl.ds(base, bpw)], n_v, sem_s),
        ]
        for cp in st:
            cp.wait()

        lane = lax.iota(jnp.int32, LANES)
        sems = [sem0, sem1]

        def _copies(c, p):
            sem = sems[p]
            cps = [
                pltpu.make_async_copy(
                    at.at[aidx.at[pl.ds(c * rows_per_chunk, rows_per_chunk)]],
                    abuf.at[p], sem)
            ]
            cps.append(pltpu.make_async_copy(
                ut.at[uidx.at[pl.ds(c * CH, CH)]], ubuf.at[p], sem))
            cps.append(pltpu.make_async_copy(
                it.at[iidx.at[pl.ds(c * CH, CH)]], ibuf.at[p], sem))
            cps.append(pltpu.make_async_copy(
                bt.at[iidx.at[pl.ds(c * CH, CH)]],
                bias_v.at[pl.ds(c * CH, CH)], sem))
            return cps

        def _fire(c, p):
            for cp in _copies(c, p):
                cp.start()

        def _drain(c, p):
            for cp in _copies(c, p):
                cp.wait()

        def _take(v, idx):
            return v.at[idx].get(mode="promise_in_bounds",
                                 unique_indices=False)

        _fire(0, 0)

        def chunk_body(ci, carry):
            for p in range(2):
                c = ci * 2 + p

                @pl.when(c + 1 < n_chunks)
                def _():
                    _fire(c + 1, 1 - p)

                _drain(c, p)

                for h in range(CH // LANES):
                    off = c * CH + h * LANES
                    n16 = n_v[pl.ds(off, LANES)]

                    def ex_body(j, ra, _h=h, _p=p):
                        x = j + _h * LANES
                        row0 = x * L
                        u = [ubuf[_p, x, pl.ds(k * LANES, LANES)]
                             for k in range(KV)]
                        acc = [abuf[_p, row0, pl.ds(k * LANES, LANES)]
                               for k in range(KV)]
                        for l in range(1, L):
                            for k in range(KV):
                                acc[k] = acc[k] + abuf[_p, row0 + l,
                                                       pl.ds(k * LANES, LANES)]
                        sa = u[0] * acc[0]
                        si = u[0] * ibuf[_p, x, pl.ds(0, LANES)]
                        for k in range(1, KV):
                            sa = sa + u[k] * acc[k]
                            si = si + u[k] * ibuf[_p, x, pl.ds(k * LANES, LANES)]
                        nj = _take(n16, jnp.full((LANES,), j, jnp.int32))
                        v = sa / nj + si
                        for sh in (8, 4, 2, 1):
                            v = v + _take(v, lane ^ sh)
                        return jnp.where(lane == j, v, ra)

                    zero = jnp.zeros((LANES,), jnp.float32)
                    ra = lax.fori_loop(0, LANES, ex_body, zero)
                    b16 = bias_v[pl.ds(off, LANES)]
                    out_v[pl.ds(off, LANES)] = ra + b16
            return carry

        lax.fori_loop(0, n_chunks // 2, chunk_body, 0)
        pltpu.sync_copy(out_v, out_hbm.at[pl.ds(base, bpw)])

    return body


def kernel(users, items, item_attributes, num_attributes, user_table,
           attr_table, item_table, item_bias_table):
    B = users.shape[0]
    L = item_attributes.shape[1]
    call = _build_sc_call(B, L)

    return call(users.astype(jnp.int32),
                items.astype(jnp.int32),
                item_attributes.astype(jnp.int32).reshape(B * L),
                num_attributes.astype(jnp.float32),
                _widen(user_table), _widen(attr_table),
                _widen(item_table),
                item_bias_table.reshape(-1))


_TR = 1024


def _widen_kernel(src, dst):
    blk = jnp.transpose(src[...])
    dst[...] = jnp.pad(blk, ((0, 0), (0, WID - EMB)))


def _widen(t):
    n = t.shape[0]
    tt = t.T
    grid = (n + _TR - 1) // _TR
    return pl.pallas_call(
        _widen_kernel,
        grid=(grid,),
        in_specs=[pl.BlockSpec((EMB, _TR), lambda i: (0, i))],
        out_specs=pl.BlockSpec((_TR, WID), lambda i: (i, 0)),
        out_shape=jax.ShapeDtypeStruct((n, WID), t.dtype),
    )(tt)


# speedup vs baseline: 1.4419x; 1.4419x over previous
import functools

import jax
import jax.numpy as jnp
from jax import lax
from jax.experimental import pallas as pl
from jax.experimental.pallas import tpu as pltpu
from jax.experimental.pallas import tpu_sc as plsc

EMB = 64
LANES = 16
NC = 2
NS = 16
NW = NC * NS


def _build_sc_call(B, L):
    assert B % (NW * 32) == 0 and EMB % LANES == 0
    bpw = B // NW
    CH = 32
    n_chunks = bpw // CH
    rows_per_chunk = CH * L
    aq = rows_per_chunk // 128
    KV = EMB // LANES

    mesh = plsc.VectorSubcoreMesh(core_axis_name="c", subcore_axis_name="s")

    @functools.partial(
        pl.kernel,
        mesh=mesh,
        compiler_params=pltpu.CompilerParams(use_tc_tiling_on_sc=False),
        out_type=jax.ShapeDtypeStruct((B,), jnp.float32),
        scratch_types=[
            pltpu.VMEM((bpw,), jnp.int32),
            pltpu.VMEM((bpw,), jnp.int32),
            pltpu.VMEM((bpw * L,), jnp.int32),
            pltpu.VMEM((bpw,), jnp.float32),
            pltpu.VMEM((bpw,), jnp.float32),
            pltpu.VMEM((2, CH, EMB), jnp.float32),
            pltpu.VMEM((2, CH, EMB), jnp.float32),
            pltpu.VMEM((2, rows_per_chunk, EMB), jnp.float32),
            pltpu.VMEM((bpw,), jnp.float32),
            pltpu.SemaphoreType.DMA,
            pltpu.SemaphoreType.DMA,
            pltpu.SemaphoreType.DMA,
        ],
    )
    def body(u_hbm, i_hbm, a_hbm, n_hbm, ut, at, it, bt,
             out_hbm, uidx, iidx, aidx, n_v, bias_v, ubuf, ibuf, abuf,
             out_v, sem0, sem1, sem_s):
        wid = lax.axis_index("s") * NC + lax.axis_index("c")
        base = wid * bpw
        st = [
            pltpu.async_copy(u_hbm.at[pl.ds(base, bpw)], uidx, sem_s),
            pltpu.async_copy(i_hbm.at[pl.ds(base, bpw)], iidx, sem_s),
            pltpu.async_copy(a_hbm.at[pl.ds(base * L, bpw * L)], aidx, sem_s),
            pltpu.async_copy(n_hbm.at[pl.ds(base, bpw)], n_v, sem_s),
        ]
        for cp in st:
            cp.wait()

        lane = lax.iota(jnp.int32, LANES)
        sems = [sem0, sem1]

        def _copies(c, p):
            sem = sems[p]
            cps = [
                pltpu.make_async_copy(
                    at.at[aidx.at[pl.ds(c * rows_per_chunk + q * 128, 128)]],
                    abuf.at[p, pl.ds(q * 128, 128), :], sem)
                for q in range(aq)
            ]
            cps.append(pltpu.make_async_copy(
                ut.at[uidx.at[pl.ds(c * CH, CH)]], ubuf.at[p], sem))
            cps.append(pltpu.make_async_copy(
                it.at[iidx.at[pl.ds(c * CH, CH)]], ibuf.at[p], sem))
            cps.append(pltpu.make_async_copy(
                bt.at[iidx.at[pl.ds(c * CH, CH)]],
                bias_v.at[pl.ds(c * CH, CH)], sem))
            return cps

        def _fire(c, p):
            for cp in _copies(c, p):
                cp.start()

        def _drain(c, p):
            for cp in _copies(c, p):
                cp.wait()

        def _take(v, idx):
            return v.at[idx].get(mode="promise_in_bounds",
                                 unique_indices=False)

        _fire(0, 0)

        def chunk_body(ci, carry):
            for p in range(2):
                c = ci * 2 + p

                @pl.when(c + 1 < n_chunks)
                def _():
                    _fire(c + 1, 1 - p)

                _drain(c, p)

                for h in range(CH // LANES):
                    off = c * CH + h * LANES
                    n16 = n_v[pl.ds(off, LANES)]

                    def ex_body(j, ra, _h=h, _p=p):
                        x = j + _h * LANES
                        row0 = x * L
                        u = [ubuf[_p, x, pl.ds(k * LANES, LANES)]
                             for k in range(KV)]
                        acc = [abuf[_p, row0, pl.ds(k * LANES, LANES)]
                               for k in range(KV)]
                        for l in range(1, L):
                            for k in range(KV):
                                acc[k] = acc[k] + abuf[_p, row0 + l,
                                                       pl.ds(k * LANES, LANES)]
                        sa = u[0] * acc[0]
                        si = u[0] * ibuf[_p, x, pl.ds(0, LANES)]
                        for k in range(1, KV):
                            sa = sa + u[k] * acc[k]
                            si = si + u[k] * ibuf[_p, x, pl.ds(k * LANES, LANES)]
                        nj = _take(n16, jnp.full((LANES,), j, jnp.int32))
                        v = sa / nj + si
                        for sh in (8, 4, 2, 1):
                            v = v + _take(v, lane ^ sh)
                        return jnp.where(lane == j, v, ra)

                    zero = jnp.zeros((LANES,), jnp.float32)
                    ra = lax.fori_loop(0, LANES, ex_body, zero)
                    b16 = bias_v[pl.ds(off, LANES)]
                    out_v[pl.ds(off, LANES)] = ra + b16
            return carry

        lax.fori_loop(0, n_chunks // 2, chunk_body, 0)
        pltpu.sync_copy(out_v, out_hbm.at[pl.ds(base, bpw)])

    return body


def kernel(users, items, item_attributes, num_attributes, user_table,
           attr_table, item_table, item_bias_table):
    B = users.shape[0]
    L = item_attributes.shape[1]
    call = _build_sc_call(B, L)

    return call(users.astype(jnp.int32),
                items.astype(jnp.int32),
                item_attributes.astype(jnp.int32).reshape(B * L),
                num_attributes.astype(jnp.float32),
                user_table, attr_table, item_table,
                item_bias_table.reshape(-1))


# speedup vs baseline: 1.5045x; 1.0435x over previous
import functools

import jax
import jax.numpy as jnp
from jax import lax
from jax.experimental import pallas as pl
from jax.experimental.pallas import tpu as pltpu
from jax.experimental.pallas import tpu_sc as plsc

EMB = 64
LANES = 16
NC = 2
NS = 16
NW = NC * NS
CH = 32


def _mesh_kernel(B, scratch):
    return functools.partial(
        pl.kernel,
        mesh=plsc.VectorSubcoreMesh(core_axis_name="c", subcore_axis_name="s"),
        compiler_params=pltpu.CompilerParams(use_tc_tiling_on_sc=False),
        out_type=jax.ShapeDtypeStruct((B,), jnp.float32),
        scratch_types=scratch,
    )


def _worker_base(bpw):
    wid = lax.axis_index("s") * NC + lax.axis_index("c")
    return wid * bpw


def _take(v, idx):
    return v.at[idx].get(mode="promise_in_bounds", unique_indices=False)


def _butterfly(v, lane):
    for sh in (8, 4, 2, 1):
        v = v + _take(v, lane ^ sh)
    return v


def _build_ui_call(B):
    bpw = B // NW
    n_chunks = bpw // CH
    KV = EMB // LANES

    @_mesh_kernel(B, [
        pltpu.VMEM((bpw,), jnp.int32),
        pltpu.VMEM((bpw,), jnp.int32),
        pltpu.VMEM((bpw,), jnp.float32),
        pltpu.VMEM((2, CH, EMB), jnp.float32),
        pltpu.VMEM((2, CH, EMB), jnp.float32),
        pltpu.VMEM((bpw,), jnp.float32),
        pltpu.SemaphoreType.DMA,
        pltpu.SemaphoreType.DMA,
        pltpu.SemaphoreType.DMA,
    ])
    def body(u_hbm, i_hbm, ut, it, bt, out_hbm,
             uidx, iidx, bias_v, ubuf, ibuf, out_v, sem0, sem1, sem_s):
        base = _worker_base(bpw)
        st = [
            pltpu.async_copy(u_hbm.at[pl.ds(base, bpw)], uidx, sem_s),
            pltpu.async_copy(i_hbm.at[pl.ds(base, bpw)], iidx, sem_s),
        ]
        for cp in st:
            cp.wait()

        lane = lax.iota(jnp.int32, LANES)
        sems = [sem0, sem1]

        def _copies(c, p):
            sem = sems[p]
            return [
                pltpu.make_async_copy(
                    ut.at[uidx.at[pl.ds(c * CH, CH)]], ubuf.at[p], sem),
                pltpu.make_async_copy(
                    it.at[iidx.at[pl.ds(c * CH, CH)]], ibuf.at[p], sem),
                pltpu.make_async_copy(
                    bt.at[iidx.at[pl.ds(c * CH, CH)]],
                    bias_v.at[pl.ds(c * CH, CH)], sem),
            ]

        def _fire(c, p):
            for cp in _copies(c, p):
                cp.start()

        def _drain(c, p):
            for cp in _copies(c, p):
                cp.wait()

        _fire(0, 0)

        def chunk_body(ci, carry):
            for p in range(2):
                c = ci * 2 + p

                @pl.when(c + 1 < n_chunks)
                def _():
                    _fire(c + 1, 1 - p)

                _drain(c, p)

                for h in range(CH // LANES):
                    off = c * CH + h * LANES

                    def ex_body(j, ra, _h=h, _p=p):
                        x = j + _h * LANES
                        si = (ubuf[_p, x, pl.ds(0, LANES)]
                              * ibuf[_p, x, pl.ds(0, LANES)])
                        for k in range(1, KV):
                            si = si + (ubuf[_p, x, pl.ds(k * LANES, LANES)]
                                       * ibuf[_p, x, pl.ds(k * LANES, LANES)])
                        v = _butterfly(si, lane)
                        return jnp.where(lane == j, v, ra)

                    zero = jnp.zeros((LANES,), jnp.float32)
                    ra = lax.fori_loop(0, LANES, ex_body, zero)
                    b16 = bias_v[pl.ds(off, LANES)]
                    out_v[pl.ds(off, LANES)] = ra + b16
            return carry

        lax.fori_loop(0, n_chunks // 2, chunk_body, 0)
        pltpu.sync_copy(out_v, out_hbm.at[pl.ds(base, bpw)])

    return body


def _build_attr_call(B, L):
    bpw = B // NW
    n_chunks = bpw // CH
    rows_per_chunk = CH * L
    aq = rows_per_chunk // 128
    KV = EMB // LANES

    @_mesh_kernel(B, [
        pltpu.VMEM((bpw,), jnp.int32),
        pltpu.VMEM((bpw * L,), jnp.int32),
        pltpu.VMEM((bpw,), jnp.float32),
        pltpu.VMEM((2, CH, EMB), jnp.float32),
        pltpu.VMEM((2, rows_per_chunk, EMB), jnp.float32),
        pltpu.VMEM((bpw,), jnp.float32),
        pltpu.SemaphoreType.DMA,
        pltpu.SemaphoreType.DMA,
        pltpu.SemaphoreType.DMA,
    ])
    def body(u_hbm, a_hbm, n_hbm, ut, at, out_hbm,
             uidx, aidx, n_v, ubuf, abuf, out_v, sem0, sem1, sem_s):
        base = _worker_base(bpw)
        st = [
            pltpu.async_copy(u_hbm.at[pl.ds(base, bpw)], uidx, sem_s),
            pltpu.async_copy(a_hbm.at[pl.ds(base * L, bpw * L)], aidx, sem_s),
            pltpu.async_copy(n_hbm.at[pl.ds(base, bpw)], n_v, sem_s),
        ]
        for cp in st:
            cp.wait()

        lane = lax.iota(jnp.int32, LANES)
        sems = [sem0, sem1]

        def _copies(c, p):
            sem = sems[p]
            cps = [
                pltpu.make_async_copy(
                    at.at[aidx.at[pl.ds(c * rows_per_chunk + q * 128, 128)]],
                    abuf.at[p, pl.ds(q * 128, 128), :], sem)
                for q in range(aq)
            ]
            cps.append(pltpu.make_async_copy(
                ut.at[uidx.at[pl.ds(c * CH, CH)]], ubuf.at[p], sem))
            return cps

        def _fire(c, p):
            for cp in _copies(c, p):
                cp.start()

        def _drain(c, p):
            for cp in _copies(c, p):
                cp.wait()

        _fire(0, 0)

        def chunk_body(ci, carry):
            for p in range(2):
                c = ci * 2 + p

                @pl.when(c + 1 < n_chunks)
                def _():
                    _fire(c + 1, 1 - p)

                _drain(c, p)

                for h in range(CH // LANES):
                    off = c * CH + h * LANES
                    n16 = n_v[pl.ds(off, LANES)]

                    def ex_body(j, ra, _h=h, _p=p):
                        x = j + _h * LANES
                        row0 = x * L
                        acc = [abuf[_p, row0, pl.ds(k * LANES, LANES)]
                               for k in range(KV)]
                        for l in range(1, L):
                            for k in range(KV):
                                acc[k] = acc[k] + abuf[_p, row0 + l,
                                                       pl.ds(k * LANES, LANES)]
                        sa = ubuf[_p, x, pl.ds(0, LANES)] * acc[0]
                        for k in range(1, KV):
                            sa = sa + (ubuf[_p, x, pl.ds(k * LANES, LANES)]
                                       * acc[k])
                        nj = _take(n16, jnp.full((LANES,), j, jnp.int32))
                        v = _butterfly(sa / nj, lane)
                        return jnp.where(lane == j, v, ra)

                    zero = jnp.zeros((LANES,), jnp.float32)
                    ra = lax.fori_loop(0, LANES, ex_body, zero)
                    out_v[pl.ds(off, LANES)] = ra
            return carry

        lax.fori_loop(0, n_chunks // 2, chunk_body, 0)
        pltpu.sync_copy(out_v, out_hbm.at[pl.ds(base, bpw)])

    return body


def kernel(users, items, item_attributes, num_attributes, user_table,
           attr_table, item_table, item_bias_table):
    B = users.shape[0]
    L = item_attributes.shape[1]
    assert B % (NW * CH) == 0 and EMB % LANES == 0

    ui = _build_ui_call(B)
    at = _build_attr_call(B, L)

    u32 = users.astype(jnp.int32)
    part_ui = ui(u32, items.astype(jnp.int32),
                 user_table, item_table, item_bias_table.reshape(-1))
    part_at = at(u32, item_attributes.astype(jnp.int32).reshape(B * L),
                 num_attributes.astype(jnp.float32),
                 user_table, attr_table)
    return part_ui + part_at
